# baseline (speedup 1.0000x reference)
import functools

import jax
import jax.numpy as jnp
from jax import lax
from jax.experimental import pallas as pl
from jax.experimental.pallas import tpu as pltpu
from jax.experimental.pallas import tpu_sc as plsc

N_TOKENS = 65536
DT, DH, DW = 344, 340, 340
DOUT = DT + DH + DW
HALF = 512
SPLIT_H = HALF - DT
SPLIT = 32768
BLK = 512


def _sc_half(th_hbm, hw_hbm, tab_th, tab_hw, out_hbm,
             idx_th, idx_hw, *bufs_and_sems, bpw, chunk, nbuf):
    rows = bufs_and_sems[:nbuf]
    gsem = bufs_and_sems[nbuf:2 * nbuf]
    wsem = bufs_and_sems[2 * nbuf:3 * nbuf]
    nc = 2
    wid = lax.axis_index("s") * nc + lax.axis_index("c")
    base = wid * bpw
    out_base = base
    nsteps = bpw // chunk

    pltpu.sync_copy(th_hbm.at[pl.ds(base, bpw)], idx_th)
    pltpu.sync_copy(hw_hbm.at[pl.ds(base, bpw)], idx_hw)

    def gather(i, b):
        sl = pl.ds(i * chunk, chunk)
        pltpu.async_copy(tab_th.at[idx_th.at[sl]],
                         rows[b].at[:, pl.ds(0, HALF)], gsem[b])
        pltpu.async_copy(tab_hw.at[idx_hw.at[sl]],
                         rows[b].at[:, pl.ds(HALF, HALF)], gsem[b])

    def wait_gather(b):
        pltpu.make_async_copy(rows[b], out_hbm.at[pl.ds(0, chunk)],
                              gsem[b]).wait()

    def write(i, b):
        pltpu.async_copy(rows[b],
                         out_hbm.at[pl.ds(out_base + i * chunk, chunk)],
                         wsem[b])

    def wait_write(b):
        pltpu.make_async_copy(rows[b], out_hbm.at[pl.ds(0, chunk)],
                              wsem[b]).wait()

    for b in range(nbuf - 1):
        gather(b, b)

    def body(i, carry):
        for b in range(nbuf):
            @pl.when((i + nbuf - 1) % nbuf == b)
            def _issue():
                @pl.when(i >= 1)
                def _drain():
                    wait_write(b)
                gather(i + nbuf - 1, b)
            @pl.when(i % nbuf == b)
            def _consume():
                wait_gather(b)
                write(i, b)
        return carry

    lax.fori_loop(0, nsteps - (nbuf - 1), body, 0)
    for j in range(nsteps - (nbuf - 1), nsteps):
        b = j % nbuf
        wait_gather(b)
        write(j, b)
    for b in range(nbuf):
        wait_write(b)


def _tc_body(ids_ref, tab_ref, out_ref):
    iota = jax.lax.broadcasted_iota(jnp.int32, (BLK, 192), 1)
    oh = ((ids_ref[0][:, None] == iota).astype(jnp.float32)
          + (ids_ref[1][:, None] == iota).astype(jnp.float32)
          + (ids_ref[2][:, None] == iota).astype(jnp.float32))
    out_ref[...] = jnp.dot(oh, tab_ref[...],
                           preferred_element_type=jnp.float32)


def kernel(position_ids, pos_embed_t, pos_embed_h, pos_embed_w):
    pid = position_ids.reshape(3, -1).astype(jnp.int32)

    th_ids = pid[0, SPLIT:] * 64 + pid[1, SPLIT:]
    hw_ids = pid[1, SPLIT:] * 64 + pid[2, SPLIT:]
    tab_th = jnp.concatenate(
        [jnp.repeat(pos_embed_t, 64, axis=0),
         jnp.tile(pos_embed_h[:, :SPLIT_H], (64, 1))], axis=-1)
    tab_hw = jnp.concatenate(
        [jnp.repeat(pos_embed_h[:, SPLIT_H:], 64, axis=0),
         jnp.tile(pos_embed_w, (64, 1))], axis=-1)

    nw = 32
    bpw = (N_TOKENS - SPLIT) // nw
    chunk = 16
    nbuf = 4

    mesh = plsc.VectorSubcoreMesh(core_axis_name="c", subcore_axis_name="s")
    sc_run = functools.partial(
        pl.kernel,
        mesh=mesh,
        out_type=jax.ShapeDtypeStruct((N_TOKENS - SPLIT, DOUT), jnp.float32),
        scratch_types=(
            [pltpu.VMEM((bpw,), jnp.int32)] * 2
            + [pltpu.VMEM((chunk, DOUT), jnp.float32)] * nbuf
            + [pltpu.SemaphoreType.DMA] * (2 * nbuf)
        ),
    )(functools.partial(_sc_half, bpw=bpw, chunk=chunk, nbuf=nbuf))
    sc_out = sc_run(th_ids, hw_ids, tab_th, tab_hw)

    shifts = jnp.array([0, 64, 128], dtype=jnp.int32)[:, None]
    ids3 = pid[:, :SPLIT] + shifts
    tab = jnp.zeros((192, DOUT), jnp.float32)
    tab = tab.at[0:64, 0:DT].set(pos_embed_t)
    tab = tab.at[64:128, DT:DT + DH].set(pos_embed_h)
    tab = tab.at[128:192, DT + DH:DOUT].set(pos_embed_w)

    tc_out = pl.pallas_call(
        _tc_body,
        grid=(SPLIT // BLK,),
        in_specs=[
            pl.BlockSpec((3, BLK), lambda i: (0, i)),
            pl.BlockSpec((192, DOUT), lambda i: (0, 0)),
        ],
        out_specs=pl.BlockSpec((BLK, DOUT), lambda i: (i, 0)),
        out_shape=jax.ShapeDtypeStruct((SPLIT, DOUT), jnp.float32),
    )(ids3, tab)
    return jnp.concatenate([tc_out, sc_out], axis=0)


# speedup vs baseline: 1.7017x; 1.7017x over previous
import functools

import jax
import jax.numpy as jnp
from jax import lax
from jax.experimental import pallas as pl
from jax.experimental.pallas import tpu as pltpu
from jax.experimental.pallas import tpu_sc as plsc

N_TOKENS = 65536
DT, DH, DW = 344, 340, 340
DOUT = DT + DH + DW
HALF = 512
SPLIT_H = HALF - DT
SPLIT = 32768
BLK = 512


def _sc_half(th_hbm, hw_hbm, tab_th, tab_hw, out_hbm,
             idx_th, idx_hw, *bufs_and_sems, bpw, chunk, nbuf):
    rows = bufs_and_sems[:nbuf]
    gsem = bufs_and_sems[nbuf:2 * nbuf]
    wsem = bufs_and_sems[2 * nbuf:3 * nbuf]
    nc = 2
    wid = lax.axis_index("s") * nc + lax.axis_index("c")
    base = wid * bpw
    out_base = SPLIT + base
    nsteps = bpw // chunk

    pltpu.sync_copy(th_hbm.at[pl.ds(base, bpw)], idx_th)
    pltpu.sync_copy(hw_hbm.at[pl.ds(base, bpw)], idx_hw)

    def gather(i, b):
        sl = pl.ds(i * chunk, chunk)
        pltpu.async_copy(tab_th.at[idx_th.at[sl]],
                         rows[b].at[:, pl.ds(0, HALF)], gsem[b])
        pltpu.async_copy(tab_hw.at[idx_hw.at[sl]],
                         rows[b].at[:, pl.ds(HALF, HALF)], gsem[b])

    def wait_gather(b):
        pltpu.make_async_copy(rows[b], out_hbm.at[pl.ds(0, chunk)],
                              gsem[b]).wait()

    def write(i, b):
        pltpu.async_copy(rows[b],
                         out_hbm.at[pl.ds(out_base + i * chunk, chunk)],
                         wsem[b])

    def wait_write(b):
        pltpu.make_async_copy(rows[b], out_hbm.at[pl.ds(0, chunk)],
                              wsem[b]).wait()

    for b in range(nbuf - 1):
        gather(b, b)

    def body(i, carry):
        for b in range(nbuf):
            @pl.when((i + nbuf - 1) % nbuf == b)
            def _issue():
                @pl.when(i >= 1)
                def _drain():
                    wait_write(b)
                gather(i + nbuf - 1, b)
            @pl.when(i % nbuf == b)
            def _consume():
                wait_gather(b)
                write(i, b)
        return carry

    lax.fori_loop(0, nsteps - (nbuf - 1), body, 0)
    for j in range(nsteps - (nbuf - 1), nsteps):
        b = j % nbuf
        wait_gather(b)
        write(j, b)
    for b in range(nbuf):
        wait_write(b)


def _tc_body(ids_ref, tab_t_ref, tab_h_ref, tab_w_ref, _sc_ref, out_ref):
    iota = jax.lax.broadcasted_iota(jnp.int32, (BLK, 64), 1)
    oh_t = (ids_ref[0][:, None] == iota).astype(jnp.float32)
    oh_h = (ids_ref[1][:, None] == iota).astype(jnp.float32)
    oh_w = (ids_ref[2][:, None] == iota).astype(jnp.float32)
    out_ref[:, 0:DT] = jnp.dot(oh_t, tab_t_ref[...],
                               preferred_element_type=jnp.float32)
    out_ref[:, DT:DT + DH] = jnp.dot(oh_h, tab_h_ref[...],
                                     preferred_element_type=jnp.float32)
    out_ref[:, DT + DH:DOUT] = jnp.dot(oh_w, tab_w_ref[...],
                                       preferred_element_type=jnp.float32)


def kernel(position_ids, pos_embed_t, pos_embed_h, pos_embed_w):
    pid = position_ids.reshape(3, -1).astype(jnp.int32)

    th_ids = pid[0, SPLIT:] * 64 + pid[1, SPLIT:]
    hw_ids = pid[1, SPLIT:] * 64 + pid[2, SPLIT:]
    tab_th = jnp.concatenate(
        [jnp.repeat(pos_embed_t, 64, axis=0),
         jnp.tile(pos_embed_h[:, :SPLIT_H], (64, 1))], axis=-1)
    tab_hw = jnp.concatenate(
        [jnp.repeat(pos_embed_h[:, SPLIT_H:], 64, axis=0),
         jnp.tile(pos_embed_w, (64, 1))], axis=-1)

    nw = 32
    bpw = (N_TOKENS - SPLIT) // nw
    chunk = 16
    nbuf = 4

    mesh = plsc.VectorSubcoreMesh(core_axis_name="c", subcore_axis_name="s")
    sc_run = functools.partial(
        pl.kernel,
        mesh=mesh,
        out_type=jax.ShapeDtypeStruct((N_TOKENS, DOUT), jnp.float32),
        scratch_types=(
            [pltpu.VMEM((bpw,), jnp.int32)] * 2
            + [pltpu.VMEM((chunk, DOUT), jnp.float32)] * nbuf
            + [pltpu.SemaphoreType.DMA] * (2 * nbuf)
        ),
    )(functools.partial(_sc_half, bpw=bpw, chunk=chunk, nbuf=nbuf))
    sc_out = sc_run(th_ids, hw_ids, tab_th, tab_hw)

    ids3 = pid[:, :SPLIT]

    return pl.pallas_call(
        _tc_body,
        grid=(SPLIT // BLK,),
        in_specs=[
            pl.BlockSpec((3, BLK), lambda i: (0, i)),
            pl.BlockSpec((64, DT), lambda i: (0, 0)),
            pl.BlockSpec((64, DH), lambda i: (0, 0)),
            pl.BlockSpec((64, DW), lambda i: (0, 0)),
            pl.BlockSpec(memory_space=pl.ANY),
        ],
        out_specs=pl.BlockSpec((BLK, DOUT), lambda i: (i, 0)),
        out_shape=jax.ShapeDtypeStruct((N_TOKENS, DOUT), jnp.float32),
        input_output_aliases={4: 0},
    )(ids3, pos_embed_t, pos_embed_h, pos_embed_w, sc_out)


# speedup vs baseline: 1.9132x; 1.1243x over previous
import functools

import jax
import jax.numpy as jnp
from jax import lax
from jax.experimental import pallas as pl
from jax.experimental.pallas import tpu as pltpu
from jax.experimental.pallas import tpu_sc as plsc

N_TOKENS = 65536
DT, DH, DW = 344, 340, 340
DOUT = DT + DH + DW
HALF = 512
SPLIT_H = HALF - DT
SPLIT = 32768
BLK = 1024


def _sc_half(th_hbm, hw_hbm, tab_th, tab_hw, out_hbm,
             idx_th, idx_hw, *bufs_and_sems, bpw, chunk, nbuf):
    rows = bufs_and_sems[:nbuf]
    gsem = bufs_and_sems[nbuf:2 * nbuf]
    wsem = bufs_and_sems[2 * nbuf:3 * nbuf]
    nc = 2
    wid = lax.axis_index("s") * nc + lax.axis_index("c")
    base = wid * bpw
    out_base = SPLIT + base
    nsteps = bpw // chunk

    pltpu.sync_copy(th_hbm.at[pl.ds(base, bpw)], idx_th)
    pltpu.sync_copy(hw_hbm.at[pl.ds(base, bpw)], idx_hw)

    def gather(i, b):
        sl = pl.ds(i * chunk, chunk)
        pltpu.async_copy(tab_th.at[idx_th.at[sl]],
                         rows[b].at[:, pl.ds(0, HALF)], gsem[b])
        pltpu.async_copy(tab_hw.at[idx_hw.at[sl]],
                         rows[b].at[:, pl.ds(HALF, HALF)], gsem[b])

    def wait_gather(b):
        pltpu.make_async_copy(rows[b], out_hbm.at[pl.ds(0, chunk)],
                              gsem[b]).wait()

    def write(i, b):
        pltpu.async_copy(rows[b],
                         out_hbm.at[pl.ds(out_base + i * chunk, chunk)],
                         wsem[b])

    def wait_write(b):
        pltpu.make_async_copy(rows[b], out_hbm.at[pl.ds(0, chunk)],
                              wsem[b]).wait()

    for b in range(nbuf - 1):
        gather(b, b)

    def body(i, carry):
        for b in range(nbuf):
            @pl.when((i + nbuf - 1) % nbuf == b)
            def _issue():
                @pl.when(i >= 1)
                def _drain():
                    wait_write(b)
                gather(i + nbuf - 1, b)
            @pl.when(i % nbuf == b)
            def _consume():
                wait_gather(b)
                write(i, b)
        return carry

    lax.fori_loop(0, nsteps - (nbuf - 1), body, 0)
    for j in range(nsteps - (nbuf - 1), nsteps):
        b = j % nbuf
        wait_gather(b)
        write(j, b)
    for b in range(nbuf):
        wait_write(b)


def _tc_body(ids_ref, tab_ref, _sc_ref, out_ref):
    iota = jax.lax.broadcasted_iota(jnp.int32, (BLK, 192), 1)
    oh = ((ids_ref[0][:, None] == iota).astype(jnp.float32)
          + (ids_ref[1][:, None] == iota).astype(jnp.float32)
          + (ids_ref[2][:, None] == iota).astype(jnp.float32))
    out_ref[...] = jnp.dot(oh, tab_ref[...],
                           preferred_element_type=jnp.float32)


def kernel(position_ids, pos_embed_t, pos_embed_h, pos_embed_w):
    pid = position_ids.reshape(3, -1).astype(jnp.int32)

    th_ids = pid[0, SPLIT:] * 64 + pid[1, SPLIT:]
    hw_ids = pid[1, SPLIT:] * 64 + pid[2, SPLIT:]
    tab_th = jnp.concatenate(
        [jnp.repeat(pos_embed_t, 64, axis=0),
         jnp.tile(pos_embed_h[:, :SPLIT_H], (64, 1))], axis=-1)
    tab_hw = jnp.concatenate(
        [jnp.repeat(pos_embed_h[:, SPLIT_H:], 64, axis=0),
         jnp.tile(pos_embed_w, (64, 1))], axis=-1)

    nw = 32
    bpw = (N_TOKENS - SPLIT) // nw
    chunk = 16
    nbuf = 4

    mesh = plsc.VectorSubcoreMesh(core_axis_name="c", subcore_axis_name="s")
    sc_run = functools.partial(
        pl.kernel,
        mesh=mesh,
        out_type=jax.ShapeDtypeStruct((N_TOKENS, DOUT), jnp.float32),
        scratch_types=(
            [pltpu.VMEM((bpw,), jnp.int32)] * 2
            + [pltpu.VMEM((chunk, DOUT), jnp.float32)] * nbuf
            + [pltpu.SemaphoreType.DMA] * (2 * nbuf)
        ),
    )(functools.partial(_sc_half, bpw=bpw, chunk=chunk, nbuf=nbuf))
    sc_out = sc_run(th_ids, hw_ids, tab_th, tab_hw)

    shifts = jnp.array([0, 64, 128], dtype=jnp.int32)[:, None]
    ids3 = pid[:, :SPLIT] + shifts
    tab = jnp.zeros((192, DOUT), jnp.float32)
    tab = tab.at[0:64, 0:DT].set(pos_embed_t)
    tab = tab.at[64:128, DT:DT + DH].set(pos_embed_h)
    tab = tab.at[128:192, DT + DH:DOUT].set(pos_embed_w)

    return pl.pallas_call(
        _tc_body,
        grid=(SPLIT // BLK,),
        in_specs=[
            pl.BlockSpec((3, BLK), lambda i: (0, i)),
            pl.BlockSpec((192, DOUT), lambda i: (0, 0)),
            pl.BlockSpec(memory_space=pl.ANY),
        ],
        out_specs=pl.BlockSpec((BLK, DOUT), lambda i: (i, 0)),
        out_shape=jax.ShapeDtypeStruct((N_TOKENS, DOUT), jnp.float32),
        input_output_aliases={2: 0},
    )(ids3, tab, sc_out)


# speedup vs baseline: 1.9601x; 1.0245x over previous
import functools

import jax
import jax.numpy as jnp
from jax import lax
from jax.experimental import pallas as pl
from jax.experimental.pallas import tpu as pltpu
from jax.experimental.pallas import tpu_sc as plsc

N_TOKENS = 65536
DT, DH, DW = 344, 340, 340
DOUT = DT + DH + DW
HALF = 512
SPLIT_H = HALF - DT
SPLIT = 32768
BLK = 2048


def _sc_half(th_hbm, hw_hbm, tab_th, tab_hw, out_hbm,
             idx_th, idx_hw, *bufs_and_sems, bpw, chunk, nbuf):
    rows = bufs_and_sems[:nbuf]
    gsem = bufs_and_sems[nbuf:2 * nbuf]
    wsem = bufs_and_sems[2 * nbuf:3 * nbuf]
    nc = 2
    wid = lax.axis_index("s") * nc + lax.axis_index("c")
    base = wid * bpw
    out_base = SPLIT + base
    nsteps = bpw // chunk

    pltpu.sync_copy(th_hbm.at[pl.ds(base, bpw)], idx_th)
    pltpu.sync_copy(hw_hbm.at[pl.ds(base, bpw)], idx_hw)

    def gather(i, b):
        sl = pl.ds(i * chunk, chunk)
        pltpu.async_copy(tab_th.at[idx_th.at[sl]],
                         rows[b].at[:, pl.ds(0, HALF)], gsem[b])
        pltpu.async_copy(tab_hw.at[idx_hw.at[sl]],
                         rows[b].at[:, pl.ds(HALF, HALF)], gsem[b])

    def wait_gather(b):
        pltpu.make_async_copy(rows[b], out_hbm.at[pl.ds(0, chunk)],
                              gsem[b]).wait()

    def write(i, b):
        pltpu.async_copy(rows[b],
                         out_hbm.at[pl.ds(out_base + i * chunk, chunk)],
                         wsem[b])

    def wait_write(b):
        pltpu.make_async_copy(rows[b], out_hbm.at[pl.ds(0, chunk)],
                              wsem[b]).wait()

    for b in range(nbuf - 1):
        gather(b, b)

    def body(i, carry):
        for b in range(nbuf):
            @pl.when((i + nbuf - 1) % nbuf == b)
            def _issue():
                @pl.when(i >= 1)
                def _drain():
                    wait_write(b)
                gather(i + nbuf - 1, b)
            @pl.when(i % nbuf == b)
            def _consume():
                wait_gather(b)
                write(i, b)
        return carry

    lax.fori_loop(0, nsteps - (nbuf - 1), body, 0)
    for j in range(nsteps - (nbuf - 1), nsteps):
        b = j % nbuf
        wait_gather(b)
        write(j, b)
    for b in range(nbuf):
        wait_write(b)


def _tc_body(ids_ref, tab_ref, _sc_ref, out_ref):
    iota = jax.lax.broadcasted_iota(jnp.int32, (BLK, 192), 1)
    oh = ((ids_ref[0][:, None] == iota).astype(jnp.float32)
          + (ids_ref[1][:, None] == iota).astype(jnp.float32)
          + (ids_ref[2][:, None] == iota).astype(jnp.float32))
    out_ref[...] = jnp.dot(oh, tab_ref[...],
                           preferred_element_type=jnp.float32)


def kernel(position_ids, pos_embed_t, pos_embed_h, pos_embed_w):
    pid = position_ids.reshape(3, -1).astype(jnp.int32)

    th_ids = pid[0, SPLIT:] * 64 + pid[1, SPLIT:]
    hw_ids = pid[1, SPLIT:] * 64 + pid[2, SPLIT:]
    tab_th = jnp.concatenate(
        [jnp.repeat(pos_embed_t, 64, axis=0),
         jnp.tile(pos_embed_h[:, :SPLIT_H], (64, 1))], axis=-1)
    tab_hw = jnp.concatenate(
        [jnp.repeat(pos_embed_h[:, SPLIT_H:], 64, axis=0),
         jnp.tile(pos_embed_w, (64, 1))], axis=-1)

    nw = 32
    bpw = (N_TOKENS - SPLIT) // nw
    chunk = 16
    nbuf = 4

    mesh = plsc.VectorSubcoreMesh(core_axis_name="c", subcore_axis_name="s")
    sc_run = functools.partial(
        pl.kernel,
        mesh=mesh,
        out_type=jax.ShapeDtypeStruct((N_TOKENS, DOUT), jnp.float32),
        scratch_types=(
            [pltpu.VMEM((bpw,), jnp.int32)] * 2
            + [pltpu.VMEM((chunk, DOUT), jnp.float32)] * nbuf
            + [pltpu.SemaphoreType.DMA] * (2 * nbuf)
        ),
    )(functools.partial(_sc_half, bpw=bpw, chunk=chunk, nbuf=nbuf))
    sc_out = sc_run(th_ids, hw_ids, tab_th, tab_hw)

    shifts = jnp.array([0, 64, 128], dtype=jnp.int32)[:, None]
    ids3 = pid[:, :SPLIT] + shifts
    tab = jnp.zeros((192, DOUT), jnp.float32)
    tab = tab.at[0:64, 0:DT].set(pos_embed_t)
    tab = tab.at[64:128, DT:DT + DH].set(pos_embed_h)
    tab = tab.at[128:192, DT + DH:DOUT].set(pos_embed_w)

    return pl.pallas_call(
        _tc_body,
        grid=(SPLIT // BLK,),
        in_specs=[
            pl.BlockSpec((3, BLK), lambda i: (0, i)),
            pl.BlockSpec((192, DOUT), lambda i: (0, 0)),
            pl.BlockSpec(memory_space=pl.ANY),
        ],
        out_specs=pl.BlockSpec((BLK, DOUT), lambda i: (i, 0)),
        out_shape=jax.ShapeDtypeStruct((N_TOKENS, DOUT), jnp.float32),
        input_output_aliases={2: 0},
    )(ids3, tab, sc_out)
